# Initial kernel scaffold; baseline (speedup 1.0000x reference)
import jax
import jax.numpy as jnp
from jax.experimental import pallas as pl


def kernel(substrate_features, substrate_edge_index, vnr_features, Wa, ba, Wc, bc, wav, bav, wcv, bcv, Wfa, bfa, Wfv, bfv):
    raise NotImplementedError("write your pallas kernel here")


# speedup vs baseline: 6.9932x; 6.9932x over previous
import jax
import jax.numpy as jnp
from jax.experimental import pallas as pl

N = 100
DIM = 128
HID = 60
ACT = 100
E = 1600


def _graph_body(edge_ref, x_ref, w_ref, b_ref, vnr_ref, wv_ref, bv_ref, out_ref):
    src = edge_ref[0:1, :]
    dst = edge_ref[1:2, :]
    ids = jax.lax.broadcasted_iota(jnp.int32, (N, E), 0)
    odst = (ids == dst).astype(jnp.float32)
    osrc = (ids == src).astype(jnp.float32)
    a = jax.lax.dot_general(odst, osrc, (((1,), (1,)), ((), ())),
                            preferred_element_type=jnp.float32)
    deg = jnp.sum(a, axis=1, keepdims=True)
    dis = jnp.where(deg > 0, jax.lax.rsqrt(jnp.maximum(deg, 1e-12)), 0.0)
    x = x_ref[...]
    hp = jax.lax.Precision.HIGHEST
    tx1 = -dis * jax.lax.dot(a, dis * x, precision=hp)
    tx2 = -2.0 * dis * jax.lax.dot(a, dis * tx1, precision=hp) - x
    t = jnp.concatenate([x, tx1, tx2], axis=1)
    g = jnp.tanh(jax.lax.dot(t, w_ref[...]) + b_ref[...])
    vvec = jax.lax.dot(vnr_ref[...], wv_ref[...]) + jnp.sum(
        bv_ref[...], axis=0, keepdims=True)
    out_ref[...] = g + vvec


def _fc_body(fa_ref, fc_ref, wfa_ref, bfa_ref, wfv_ref, bfv_ref, lo_ref, vo_ref):
    lo_ref[...] = jax.lax.dot(fa_ref[...], wfa_ref[...]) + bfa_ref[...]
    vo_ref[...] = jax.lax.dot(fc_ref[...], wfv_ref[...]) + bfv_ref[...]


def kernel(substrate_features, substrate_edge_index, vnr_features,
           Wa, ba, Wc, bc, wav, bav, wcv, bcv, Wfa, bfa, Wfv, bfv):
    edge = substrate_edge_index.astype(jnp.int32)
    w_stack = jnp.concatenate([Wa, Wc], axis=2).reshape(3 * DIM, 2 * HID)
    b_all = jnp.concatenate([ba, bc]).reshape(1, 2 * HID)
    wv_all = jnp.concatenate([wav.reshape(3, HID), wcv.reshape(3, HID)], axis=1)
    bv_all = jnp.concatenate([bav, bcv], axis=1)

    final = pl.pallas_call(
        _graph_body,
        out_shape=jax.ShapeDtypeStruct((N, 2 * HID), jnp.float32),
    )(edge, substrate_features, w_stack, b_all, vnr_features, wv_all, bv_all)

    flat_a = final[:, :HID].reshape(1, N * HID)
    flat_c = final[:, HID:].reshape(1, N * HID)
    logits, values = pl.pallas_call(
        _fc_body,
        out_shape=(jax.ShapeDtypeStruct((1, ACT), jnp.float32),
                   jax.ShapeDtypeStruct((1, 1), jnp.float32)),
    )(flat_a, flat_c, Wfa, bfa.reshape(1, ACT), Wfv, bfv.reshape(1, 1))
    return logits, values


# speedup vs baseline: 10.6224x; 1.5190x over previous
import jax
import jax.numpy as jnp
from jax.experimental import pallas as pl

N = 100
DIM = 128
HID = 60
ACT = 100
E = 1600


def _body(edge_ref, x_ref, vnr_ref,
          wa_ref, ba_ref, wc_ref, bc_ref,
          wav_ref, bav_ref, wcv_ref, bcv_ref,
          wfa_ref, bfa_ref, wfv_ref, bfv_ref,
          lo_ref, vo_ref):
    src = edge_ref[0:1, :]
    dst = edge_ref[1:2, :]
    ids = jax.lax.broadcasted_iota(jnp.int32, (N, E), 0)
    odst = (ids == dst).astype(jnp.float32)
    osrc = (ids == src).astype(jnp.float32)
    a = jax.lax.dot_general(odst, osrc, (((1,), (1,)), ((), ())),
                            preferred_element_type=jnp.float32)
    deg = jnp.sum(a, axis=1, keepdims=True)
    dis = jnp.where(deg > 0, jax.lax.rsqrt(jnp.maximum(deg, 1e-12)), 0.0)
    x = x_ref[...]
    hp = jax.lax.Precision.HIGHEST
    tx1 = -dis * jax.lax.dot(a, dis * x, precision=hp)
    tx2 = -2.0 * dis * jax.lax.dot(a, dis * tx1, precision=hp) - x
    vnr = vnr_ref[...]

    def branch(w3, b, wv, bv):
        g = jnp.tanh(jax.lax.dot(x, w3[0]) + jax.lax.dot(tx1, w3[1]) +
                     jax.lax.dot(tx2, w3[2]) + b)
        vvec = jax.lax.dot(vnr, wv) + jnp.sum(bv, axis=0, keepdims=True)
        return g + vvec

    fa = branch(wa_ref[...], ba_ref[...], wav_ref[...], bav_ref[...])
    fc = branch(wc_ref[...], bc_ref[...], wcv_ref[...], bcv_ref[...])

    c = jax.lax.dot_general(fa, wfa_ref[...], (((0,), (0,)), ((), ())),
                            preferred_element_type=jnp.float32)
    acc = bfa_ref[...]
    for h in range(HID):
        acc = acc + c[h:h + 1, h * ACT:(h + 1) * ACT]
    lo_ref[...] = acc

    vo_ref[...] = (jnp.sum(fc * wfv_ref[...]) + bfv_ref[0, 0]).reshape(1, 1)


def kernel(substrate_features, substrate_edge_index, vnr_features,
           Wa, ba, Wc, bc, wav, bav, wcv, bcv, Wfa, bfa, Wfv, bfv):
    logits, values = pl.pallas_call(
        _body,
        out_shape=(jax.ShapeDtypeStruct((1, ACT), jnp.float32),
                   jax.ShapeDtypeStruct((1, 1), jnp.float32)),
    )(substrate_edge_index.astype(jnp.int32), substrate_features, vnr_features,
      Wa, ba.reshape(1, HID), Wc, bc.reshape(1, HID),
      wav.reshape(3, HID), bav, wcv.reshape(3, HID), bcv,
      Wfa.reshape(N, HID * ACT), bfa.reshape(1, ACT),
      Wfv.reshape(N, HID), bfv.reshape(1, 1))
    return logits, values


# speedup vs baseline: 10.9606x; 1.0318x over previous
import jax
import jax.numpy as jnp
from jax.experimental import pallas as pl
from jax.experimental.pallas import tpu as pltpu

N = 100
DIM = 128
HID = 60
ACT = 100
E = 1600


def _body(edge_ref, x_ref, vnr_ref,
          wa_ref, ba_ref, wc_ref, bc_ref,
          wav_ref, bav_ref, wcv_ref, bcv_ref,
          wfa_ref, bfa_ref, wfv_ref, bfv_ref,
          lo_ref, vo_ref, flat_ref):
    src = edge_ref[0:1, :]
    dst = edge_ref[1:2, :]
    ids = jax.lax.broadcasted_iota(jnp.int32, (N, E), 0)
    odst = (ids == dst).astype(jnp.float32)
    osrc = (ids == src).astype(jnp.float32)
    a = jax.lax.dot_general(odst, osrc, (((1,), (1,)), ((), ())),
                            preferred_element_type=jnp.float32)
    deg = jnp.sum(a, axis=1, keepdims=True)
    dis = jnp.where(deg > 0, jax.lax.rsqrt(jnp.maximum(deg, 1e-12)), 0.0)
    x = x_ref[...]
    hp = jax.lax.Precision.HIGHEST
    tx1 = -dis * jax.lax.dot(a, dis * x, precision=hp)
    tx2 = -2.0 * dis * jax.lax.dot(a, dis * tx1, precision=hp) - x
    vnr = vnr_ref[...]

    def branch(w3, b, wv, bv):
        g = jnp.tanh(jax.lax.dot(x, w3[0]) + jax.lax.dot(tx1, w3[1]) +
                     jax.lax.dot(tx2, w3[2]) + b.reshape(1, HID))
        vvec = (vnr[0, 0] * wv[0] + vnr[0, 1] * wv[1] + vnr[0, 2] * wv[2]
                + jnp.sum(bv, axis=0, keepdims=True))
        return g + vvec

    fa = branch(wa_ref[...], ba_ref[...], wav_ref[...], bav_ref[...])
    fc = branch(wc_ref[...], bc_ref[...], wcv_ref[...], bcv_ref[...])

    for n in range(N):
        flat_ref[:, n * HID:(n + 1) * HID] = fa[n:n + 1, :]
    flat_a = flat_ref[...]
    lo_ref[...] = jax.lax.dot(flat_a, wfa_ref[...]) + bfa_ref[...].reshape(1, ACT)
    vo_ref[...] = (jnp.sum(fc * wfv_ref[...]) + bfv_ref[0]).reshape(1, 1)


def kernel(substrate_features, substrate_edge_index, vnr_features,
           Wa, ba, Wc, bc, wav, bav, wcv, bcv, Wfa, bfa, Wfv, bfv):
    logits, values = pl.pallas_call(
        _body,
        out_shape=(jax.ShapeDtypeStruct((1, ACT), jnp.float32),
                   jax.ShapeDtypeStruct((1, 1), jnp.float32)),
        scratch_shapes=[pltpu.VMEM((1, N * HID), jnp.float32)],
    )(substrate_edge_index.astype(jnp.int32), substrate_features, vnr_features,
      Wa, ba, Wc, bc, wav, bav, wcv, bcv, Wfa, bfa, Wfv.reshape(N, HID), bfv)
    return logits, values
